# baseline (speedup 1.0000x reference)
import functools

import jax
import jax.numpy as jnp
from jax import lax
from jax.experimental import pallas as pl
from jax.experimental.pallas import tpu as pltpu
from jax.experimental.pallas import tpu_sc as plsc

_NUM_CORES = 2
_NUM_SUBCORES = 16
_NUM_WORKERS = _NUM_CORES * _NUM_SUBCORES
_CHUNK = 128
_LANES = 16
_COLS_PER_TILE = 26


def _compact_body(n_rows, dim, tv_hbm, out_hbm, stg0, stg1, outb0, outb1,
                  sem_i0, sem_i1, sem_o0, sem_o1):
    wid = lax.axis_index("s") * _NUM_CORES + lax.axis_index("c")
    t0 = wid * _COLS_PER_TILE
    iota = lax.iota(jnp.int32, _LANES)
    stgs = [stg0, stg1]
    outbs = [outb0, outb1]
    sem_in = [sem_i0, sem_i1]
    sem_out = [sem_o0, sem_o1]
    max_start = (n_rows // _CHUNK - 1) * _CHUNK
    blk_words = _CHUNK * dim
    n_v = blk_words // _LANES
    per_row = dim // _LANES

    def src_start(tc):
        return pl.multiple_of(jnp.minimum(tc * _CHUNK, max_start), _CHUNK)

    def fire(tc, b):
        pltpu.async_copy(
            tv_hbm.at[:, pl.ds(src_start(tc), _CHUNK)], stgs[b], sem_in[b]
        )

    def wait_in(b):
        pltpu.make_async_copy(
            tv_hbm.at[:, pl.ds(0, _CHUNK)], stgs[b], sem_in[b]
        ).wait()

    def wait_out(b):
        pltpu.make_async_copy(
            out_hbm.at[pl.ds(0, blk_words)], outbs[b], sem_out[b]
        ).wait()

    fire(t0, 0)

    def outer(i2, carry):
        for b in range(2):
            i = 2 * i2 + b
            tc = t0 + i
            fire(tc + 1, (b + 1) % 2)
            wait_in(b)

            @pl.when(i2 >= 1)
            def _():
                wait_out(b)

            for v0 in range(0, n_v, 8):
                vals = [
                    plsc.load_gather(
                        stgs[b],
                        [iota + _LANES * ((v0 + u) % per_row),
                         jnp.full((_LANES,), (v0 + u) // per_row, jnp.int32)],
                    )
                    for u in range(8)
                ]
                for u in range(8):
                    outbs[b][pl.ds((v0 + u) * _LANES, _LANES)] = vals[u]
            pltpu.async_copy(
                outbs[b],
                out_hbm.at[pl.ds(src_start(tc) * dim, blk_words)],
                sem_out[b],
            )
        return carry

    lax.fori_loop(0, _COLS_PER_TILE // 2, outer, 0)
    wait_in(0)
    wait_out(0)
    wait_out(1)


def _gather_body(n_chunks, dim, tail_start, idx_hbm, table_hbm, tail_hbm,
                 out_hbm, idx_v, rows_v, tail_v, buf_v, sems):
    wid = lax.axis_index("s") * _NUM_CORES + lax.axis_index("c")
    pltpu.sync_copy(idx_hbm.at[wid], idx_v)
    pltpu.sync_copy(tail_hbm, tail_v)
    copies = [
        pltpu.async_copy(
            table_hbm.at[idx_v.at[j]], rows_v.at[pl.ds(j * _CHUNK, _CHUNK)],
            sems[j],
        )
        for j in range(n_chunks)
    ]
    iota = lax.iota(jnp.int32, _LANES)
    n_lc = _CHUNK // _LANES
    for j in range(n_chunks):
        copies[j].wait()
        idx0 = []
        masks = []
        tidx = []
        for lc in range(n_lc):
            idx0.append(
                jnp.full((_LANES,), j * _CHUNK + lc * _LANES, jnp.int32) + iota
            )
            ids = idx_v[j, pl.ds(lc * _LANES, _LANES)]
            masks.append(ids >= tail_start)
            tidx.append(jnp.maximum(ids - tail_start, 0))
        for c in range(dim):
            cvec = jnp.full((_LANES,), c, jnp.int32)
            vals = [
                jnp.where(
                    masks[lc],
                    plsc.load_gather(tail_v, [tidx[lc], cvec]),
                    plsc.load_gather(rows_v, [idx0[lc], cvec]),
                )
                for lc in range(n_lc)
            ]
            for lc in range(n_lc):
                buf_v[c // 8, j, c % 8, pl.ds(lc * _LANES, _LANES)] = vals[lc]
    j0 = wid * n_chunks
    for i in range(dim // 8):
        pltpu.sync_copy(buf_v.at[i], out_hbm.at[i, pl.ds(j0, n_chunks)])


def kernel(fuel_id, table):
    (batch,) = fuel_id.shape
    n_rows, dim = table.shape
    b_per_w = batch // _NUM_WORKERS
    n_chunks = b_per_w // _CHUNK
    tail_start = (n_rows // _CHUNK) * _CHUNK
    n_tail = n_rows - tail_start
    idx = fuel_id.astype(jnp.int32).reshape(_NUM_WORKERS, n_chunks, _CHUNK)
    tail = table[tail_start:, :]
    mesh = plsc.VectorSubcoreMesh(core_axis_name="c", subcore_axis_name="s")

    compact = pl.kernel(
        functools.partial(_compact_body, n_rows, dim),
        out_type=jax.ShapeDtypeStruct((n_rows * dim,), jnp.float32),
        mesh=mesh,
        scratch_types=[
            pltpu.VMEM((dim, _CHUNK), jnp.float32),
            pltpu.VMEM((dim, _CHUNK), jnp.float32),
            pltpu.VMEM((_CHUNK * dim,), jnp.float32),
            pltpu.VMEM((_CHUNK * dim,), jnp.float32),
            pltpu.SemaphoreType.DMA,
            pltpu.SemaphoreType.DMA,
            pltpu.SemaphoreType.DMA,
            pltpu.SemaphoreType.DMA,
        ],
        compiler_params=pltpu.CompilerParams(
            use_tc_tiling_on_sc=True, needs_layout_passes=False
        ),
    )
    t_lin = compact(table.T).reshape(n_rows, dim)

    gather = pl.kernel(
        functools.partial(_gather_body, n_chunks, dim, tail_start),
        out_type=jax.ShapeDtypeStruct(
            (dim // 8, batch // _CHUNK, 8, _CHUNK), jnp.float32
        ),
        mesh=mesh,
        scratch_types=[
            pltpu.VMEM((n_chunks, _CHUNK), jnp.int32),
            pltpu.VMEM((b_per_w, dim), jnp.float32),
            pltpu.VMEM((n_tail, dim), jnp.float32),
            pltpu.VMEM((dim // 8, n_chunks, 8, _CHUNK), jnp.float32),
            [pltpu.SemaphoreType.DMA] * n_chunks,
        ],
        compiler_params=pltpu.CompilerParams(
            use_tc_tiling_on_sc=False, needs_layout_passes=False
        ),
    )
    out4d = gather(idx, t_lin, tail)
    return out4d.transpose(1, 3, 0, 2).reshape(batch, dim)


# speedup vs baseline: 2.2855x; 2.2855x over previous
import functools

import jax
import jax.numpy as jnp
from jax import lax
from jax.experimental import pallas as pl
from jax.experimental.pallas import tpu as pltpu
from jax.experimental.pallas import tpu_sc as plsc

_NUM_CORES = 2
_NUM_SUBCORES = 16
_NUM_WORKERS = _NUM_CORES * _NUM_SUBCORES
_CHUNK = 128
_LANES = 16
_COLS_PER_TILE = 26


def _compact_body(n_rows, dim, tv_hbm, out_hbm, stg0, stg1, outb0, outb1,
                  sem_i0, sem_i1, sem_o0, sem_o1):
    wid = lax.axis_index("s") * _NUM_CORES + lax.axis_index("c")
    t0 = wid * _COLS_PER_TILE
    iota = lax.iota(jnp.int32, _LANES)
    stgs = [stg0, stg1]
    outbs = [outb0, outb1]
    sem_in = [sem_i0, sem_i1]
    sem_out = [sem_o0, sem_o1]
    max_start = (n_rows // _CHUNK - 1) * _CHUNK
    blk_words = _CHUNK * dim
    n_v = blk_words // _LANES
    per_row = dim // _LANES

    def src_start(tc):
        return pl.multiple_of(jnp.minimum(tc * _CHUNK, max_start), _CHUNK)

    def fire(tc, b):
        pltpu.async_copy(
            tv_hbm.at[:, pl.ds(src_start(tc), _CHUNK)], stgs[b], sem_in[b]
        )

    def wait_in(b):
        pltpu.make_async_copy(
            tv_hbm.at[:, pl.ds(0, _CHUNK)], stgs[b], sem_in[b]
        ).wait()

    def wait_out(b):
        pltpu.make_async_copy(
            out_hbm.at[pl.ds(0, blk_words)], outbs[b], sem_out[b]
        ).wait()

    fire(t0, 0)

    def outer(i2, carry):
        for b in range(2):
            i = 2 * i2 + b
            tc = t0 + i
            fire(tc + 1, (b + 1) % 2)
            wait_in(b)

            @pl.when(i2 >= 1)
            def _():
                wait_out(b)

            stg_b = stgs[b]
            outb_b = outbs[b]
            iotas = [iota + _LANES * lc for lc in range(per_row)]

            @functools.partial(plsc.parallel_loop, 0, _CHUNK, unroll=8)
            def _(rr):
                rvec = lax.broadcast(rr, (_LANES,))
                for lc in range(per_row):
                    outb_b[pl.ds(rr * dim + lc * _LANES, _LANES)] = (
                        plsc.load_gather(stg_b, [iotas[lc], rvec])
                    )
            pltpu.async_copy(
                outbs[b],
                out_hbm.at[pl.ds(src_start(tc) * dim, blk_words)],
                sem_out[b],
            )
        return carry

    lax.fori_loop(0, _COLS_PER_TILE // 2, outer, 0)
    wait_in(0)
    wait_out(0)
    wait_out(1)


def _gather_body(n_chunks, dim, tail_start, idx_hbm, table_hbm, tail_hbm,
                 out_hbm, idx_v, rows_v, tail_v, buf_v, sems):
    wid = lax.axis_index("s") * _NUM_CORES + lax.axis_index("c")
    pltpu.sync_copy(idx_hbm.at[wid], idx_v)
    pltpu.sync_copy(tail_hbm, tail_v)
    copies = [
        pltpu.async_copy(
            table_hbm.at[idx_v.at[j]], rows_v.at[pl.ds(j * _CHUNK, _CHUNK)],
            sems[j],
        )
        for j in range(n_chunks)
    ]
    iota = lax.iota(jnp.int32, _LANES)
    n_lc = _CHUNK // _LANES
    for j in range(n_chunks):
        copies[j].wait()
        idx0 = []
        masks = []
        tidx = []
        for lc in range(n_lc):
            idx0.append(
                jnp.full((_LANES,), j * _CHUNK + lc * _LANES, jnp.int32) + iota
            )
            ids = idx_v[j, pl.ds(lc * _LANES, _LANES)]
            masks.append(ids >= tail_start)
            tidx.append(jnp.maximum(ids - tail_start, 0))
        @functools.partial(plsc.parallel_loop, 0, dim, unroll=4)
        def _(c):
            cvec = lax.broadcast(c, (_LANES,))
            for lc in range(n_lc):
                v = jnp.where(
                    masks[lc],
                    plsc.load_gather(tail_v, [tidx[lc], cvec]),
                    plsc.load_gather(rows_v, [idx0[lc], cvec]),
                )
                buf_v[c // 8, j, c % 8, pl.ds(lc * _LANES, _LANES)] = v
    j0 = wid * n_chunks
    for i in range(dim // 8):
        pltpu.sync_copy(buf_v.at[i], out_hbm.at[i, pl.ds(j0, n_chunks)])


def kernel(fuel_id, table):
    (batch,) = fuel_id.shape
    n_rows, dim = table.shape
    b_per_w = batch // _NUM_WORKERS
    n_chunks = b_per_w // _CHUNK
    tail_start = (n_rows // _CHUNK) * _CHUNK
    n_tail = n_rows - tail_start
    idx = fuel_id.astype(jnp.int32).reshape(_NUM_WORKERS, n_chunks, _CHUNK)
    tail = table[tail_start:, :]
    mesh = plsc.VectorSubcoreMesh(core_axis_name="c", subcore_axis_name="s")

    compact = pl.kernel(
        functools.partial(_compact_body, n_rows, dim),
        out_type=jax.ShapeDtypeStruct((n_rows * dim,), jnp.float32),
        mesh=mesh,
        scratch_types=[
            pltpu.VMEM((dim, _CHUNK), jnp.float32),
            pltpu.VMEM((dim, _CHUNK), jnp.float32),
            pltpu.VMEM((_CHUNK * dim,), jnp.float32),
            pltpu.VMEM((_CHUNK * dim,), jnp.float32),
            pltpu.SemaphoreType.DMA,
            pltpu.SemaphoreType.DMA,
            pltpu.SemaphoreType.DMA,
            pltpu.SemaphoreType.DMA,
        ],
        compiler_params=pltpu.CompilerParams(
            use_tc_tiling_on_sc=True, needs_layout_passes=False
        ),
    )
    t_lin = compact(table.T).reshape(n_rows, dim)

    gather = pl.kernel(
        functools.partial(_gather_body, n_chunks, dim, tail_start),
        out_type=jax.ShapeDtypeStruct(
            (dim // 8, batch // _CHUNK, 8, _CHUNK), jnp.float32
        ),
        mesh=mesh,
        scratch_types=[
            pltpu.VMEM((n_chunks, _CHUNK), jnp.int32),
            pltpu.VMEM((b_per_w, dim), jnp.float32),
            pltpu.VMEM((n_tail, dim), jnp.float32),
            pltpu.VMEM((dim // 8, n_chunks, 8, _CHUNK), jnp.float32),
            [pltpu.SemaphoreType.DMA] * n_chunks,
        ],
        compiler_params=pltpu.CompilerParams(
            use_tc_tiling_on_sc=False, needs_layout_passes=False
        ),
    )
    out4d = gather(idx, t_lin, tail)
    return out4d.transpose(1, 3, 0, 2).reshape(batch, dim)
